# baseline (speedup 1.0000x reference)
import functools

import jax
import jax.numpy as jnp
from jax import lax
from jax.experimental import pallas as pl
from jax.experimental.pallas import tpu as pltpu
from jax.experimental.pallas import tpu_sc as plsc

NUM_EMB = 1_000_000
DIM = 32
BATCH = 16384
N_FIELDS = 26

NC = 2
NS = 16
NW = NC * NS
ROWS_PER_TILE = BATCH // NW
G = 64
NGROUP = ROWS_PER_TILE // G

_mesh = plsc.VectorSubcoreMesh(core_axis_name="c", subcore_axis_name="s")


@functools.partial(
    pl.kernel,
    mesh=_mesh,
    out_type=jax.ShapeDtypeStruct((BATCH, N_FIELDS, DIM), jnp.float32),
    compiler_params=pltpu.CompilerParams(use_tc_tiling_on_sc=False),
    scratch_types=[
        pltpu.VMEM((ROWS_PER_TILE, N_FIELDS), jnp.int32),
        pltpu.VMEM((2, G, N_FIELDS, DIM), jnp.float32),
        pltpu.SemaphoreType.DMA,
        pltpu.SemaphoreType.DMA,
        pltpu.SemaphoreType.DMA,
        pltpu.SemaphoreType.DMA,
    ],
)
def _gather_sc(idx_hbm, table_hbm, out_hbm, idx_v, buf, gsem0, gsem1,
               ssem0, ssem1):
    wid = lax.axis_index("s") * NC + lax.axis_index("c")
    base = wid * ROWS_PER_TILE
    gsems = (gsem0, gsem1)
    ssems = (ssem0, ssem1)

    pltpu.sync_copy(idx_hbm.at[pl.ds(base, ROWS_PER_TILE)], idx_v)

    def start_gathers(g, h):
        def body(r, c):
            pltpu.async_copy(
                table_hbm.at[idx_v.at[g * G + r]],
                buf.at[h].at[r],
                gsems[h],
            )
            return c
        lax.fori_loop(0, G, body, 0)

    def wait_gathers(h):
        pltpu.make_async_copy(out_hbm.at[pl.ds(0, G)], buf.at[h],
                              gsems[h]).wait()

    def start_scatter(g, h):
        pltpu.async_copy(buf.at[h], out_hbm.at[pl.ds(base + g * G, G)],
                         ssems[h])

    def wait_scatter(h):
        pltpu.make_async_copy(buf.at[h], out_hbm.at[pl.ds(0, G)],
                              ssems[h]).wait()

    start_gathers(0, 0)
    for g in range(NGROUP):
        h = g % 2
        if g + 1 < NGROUP:
            if g >= 1:
                wait_scatter(1 - h)
            start_gathers(g + 1, 1 - h)
        wait_gathers(h)
        start_scatter(g, h)
    wait_scatter(0)
    wait_scatter(1)


def kernel(indices, table):
    idx = jnp.minimum(indices.astype(jnp.int32), NUM_EMB - 1)
    out = _gather_sc(idx, table)
    return jnp.maximum(out, jnp.float32(-3.4e38))


# speedup vs baseline: 1.1967x; 1.1967x over previous
import functools

import jax
import jax.numpy as jnp
from jax import lax
from jax.experimental import pallas as pl
from jax.experimental.pallas import tpu as pltpu
from jax.experimental.pallas import tpu_sc as plsc

NUM_EMB = 1_000_000
DIM = 32
BATCH = 16384
N_FIELDS = 26
PAD_FIELDS = 32

NC = 2
NS = 16
NW = NC * NS
ROWS_PER_TILE = BATCH // NW
G = 32
NGROUP = ROWS_PER_TILE // G

_mesh = plsc.VectorSubcoreMesh(core_axis_name="c", subcore_axis_name="s")


@functools.partial(
    pl.kernel,
    mesh=_mesh,
    out_type=jax.ShapeDtypeStruct((BATCH, PAD_FIELDS, DIM), jnp.float32),
    compiler_params=pltpu.CompilerParams(use_tc_tiling_on_sc=False),
    scratch_types=[
        pltpu.VMEM((ROWS_PER_TILE, N_FIELDS), jnp.int32),
        pltpu.VMEM((2, G, PAD_FIELDS, DIM), jnp.float32),
        pltpu.SemaphoreType.DMA,
        pltpu.SemaphoreType.DMA,
        pltpu.SemaphoreType.DMA,
        pltpu.SemaphoreType.DMA,
    ],
)
def _gather_sc(idx_hbm, table_hbm, out_hbm, idx_v, buf, gsem0, gsem1,
               ssem0, ssem1):
    wid = lax.axis_index("s") * NC + lax.axis_index("c")
    base = wid * ROWS_PER_TILE
    gsems = (gsem0, gsem1)
    ssems = (ssem0, ssem1)

    pltpu.sync_copy(idx_hbm.at[pl.ds(base, ROWS_PER_TILE)], idx_v)

    def start_gathers(g, h):
        def body(r, c):
            pltpu.async_copy(
                table_hbm.at[idx_v.at[g * G + r]],
                buf.at[h].at[r].at[pl.ds(0, N_FIELDS)],
                gsems[h],
            )
            return c
        lax.fori_loop(0, G, body, 0)

    def wait_gathers(h):
        pltpu.make_async_copy(out_hbm.at[pl.ds(0, N_FIELDS)],
                              buf.at[h].at[pl.ds(0, N_FIELDS)],
                              gsems[h]).wait()

    def start_scatter(g, h):
        pltpu.async_copy(buf.at[h], out_hbm.at[pl.ds(base + g * G, G)],
                         ssems[h])

    def wait_scatter(h):
        pltpu.make_async_copy(buf.at[h], out_hbm.at[pl.ds(0, G)],
                              ssems[h]).wait()

    start_gathers(0, 0)
    for g in range(NGROUP):
        h = g % 2
        if g + 1 < NGROUP:
            if g >= 1:
                wait_scatter(1 - h)
            start_gathers(g + 1, 1 - h)
        wait_gathers(h)
        start_scatter(g, h)
    wait_scatter(0)
    wait_scatter(1)


def kernel(indices, table):
    out = _gather_sc(indices.astype(jnp.int32), table)
    return out[:, :N_FIELDS, :]


# speedup vs baseline: 1.2141x; 1.0145x over previous
import functools

import jax
import jax.numpy as jnp
from jax import lax
from jax.experimental import pallas as pl
from jax.experimental.pallas import tpu as pltpu
from jax.experimental.pallas import tpu_sc as plsc

NUM_EMB = 1_000_000
DIM = 32
BATCH = 16384
N_FIELDS = 26
PAD_FIELDS = 32

NC = 2
NS = 16
NW = NC * NS
ROWS_PER_TILE = BATCH // NW
G = 64
NGROUP = ROWS_PER_TILE // G

_mesh = plsc.VectorSubcoreMesh(core_axis_name="c", subcore_axis_name="s")


@functools.partial(
    pl.kernel,
    mesh=_mesh,
    out_type=jax.ShapeDtypeStruct((BATCH, N_FIELDS, DIM), jnp.float32),
    compiler_params=pltpu.CompilerParams(use_tc_tiling_on_sc=False),
    scratch_types=[
        pltpu.VMEM((ROWS_PER_TILE, PAD_FIELDS), jnp.int32),
        pltpu.VMEM((2, G, N_FIELDS, DIM), jnp.float32),
        pltpu.SemaphoreType.DMA,
        pltpu.SemaphoreType.DMA,
        pltpu.SemaphoreType.DMA,
        pltpu.SemaphoreType.DMA,
    ],
)
def _gather_sc(idx_hbm, table_hbm, out_hbm, idx_v, buf, gsem0, gsem1,
               ssem0, ssem1):
    wid = lax.axis_index("s") * NC + lax.axis_index("c")
    base = wid * ROWS_PER_TILE
    gsems = (gsem0, gsem1)
    ssems = (ssem0, ssem1)

    pltpu.sync_copy(idx_hbm.at[pl.ds(base, ROWS_PER_TILE)], idx_v)

    def start_gathers(g, h):
        def body(r, c):
            pltpu.async_copy(
                table_hbm.at[idx_v.at[g * G + r].at[pl.ds(0, N_FIELDS)]],
                buf.at[h].at[r],
                gsems[h],
            )
            return c
        lax.fori_loop(0, G, body, 0)

    def wait_gathers(h):
        pltpu.make_async_copy(out_hbm.at[pl.ds(0, G)], buf.at[h],
                              gsems[h]).wait()

    def start_scatter(g, h):
        pltpu.async_copy(buf.at[h], out_hbm.at[pl.ds(base + g * G, G)],
                         ssems[h])

    def wait_scatter(h):
        pltpu.make_async_copy(buf.at[h], out_hbm.at[pl.ds(0, G)],
                              ssems[h]).wait()

    start_gathers(0, 0)
    for g in range(NGROUP):
        h = g % 2
        if g + 1 < NGROUP:
            if g >= 1:
                wait_scatter(1 - h)
            start_gathers(g + 1, 1 - h)
        wait_gathers(h)
        start_scatter(g, h)
    wait_scatter(0)
    wait_scatter(1)


def kernel(indices, table):
    idx_pad = jnp.pad(indices.astype(jnp.int32), ((0, 0), (0, PAD_FIELDS - N_FIELDS)))
    return _gather_sc(idx_pad, table)


# speedup vs baseline: 1.2172x; 1.0026x over previous
import functools

import jax
import jax.numpy as jnp
from jax import lax
from jax.experimental import pallas as pl
from jax.experimental.pallas import tpu as pltpu
from jax.experimental.pallas import tpu_sc as plsc

NUM_EMB = 1_000_000
DIM = 32
BATCH = 16384
N_FIELDS = 26
TOTAL = BATCH * N_FIELDS

NC = 2
NS = 16
NW = NC * NS
IDX_PER_TILE = TOTAL // NW
CHUNK = 128
G = 1664
NCHUNK = G // CHUNK
NGROUP = IDX_PER_TILE // G

_mesh = plsc.VectorSubcoreMesh(core_axis_name="c", subcore_axis_name="s")


@functools.partial(
    pl.kernel,
    mesh=_mesh,
    out_type=jax.ShapeDtypeStruct((TOTAL, DIM), jnp.float32),
    compiler_params=pltpu.CompilerParams(use_tc_tiling_on_sc=False),
    scratch_types=[
        pltpu.VMEM((IDX_PER_TILE,), jnp.int32),
        pltpu.VMEM((2, G, DIM), jnp.float32),
        pltpu.SemaphoreType.DMA,
        pltpu.SemaphoreType.DMA,
        pltpu.SemaphoreType.DMA,
        pltpu.SemaphoreType.DMA,
    ],
)
def _gather_sc(idx_hbm, table_hbm, out_hbm, idx_v, buf, gsem0, gsem1,
               ssem0, ssem1):
    wid = lax.axis_index("s") * NC + lax.axis_index("c")
    base = wid * IDX_PER_TILE
    gsems = (gsem0, gsem1)
    ssems = (ssem0, ssem1)

    pltpu.sync_copy(idx_hbm.at[pl.ds(base, IDX_PER_TILE)], idx_v)

    def start_gathers(g, h):
        def body(c, carry):
            pltpu.async_copy(
                table_hbm.at[idx_v.at[pl.ds(g * G + c * CHUNK, CHUNK)]],
                buf.at[h].at[pl.ds(c * CHUNK, CHUNK)],
                gsems[h],
            )
            return carry
        lax.fori_loop(0, NCHUNK, body, 0)

    def wait_gathers(h):
        pltpu.make_async_copy(out_hbm.at[pl.ds(0, G)], buf.at[h],
                              gsems[h]).wait()

    def start_scatter(g, h):
        pltpu.async_copy(buf.at[h], out_hbm.at[pl.ds(base + g * G, G)],
                         ssems[h])

    def wait_scatter(h):
        pltpu.make_async_copy(buf.at[h], out_hbm.at[pl.ds(0, G)],
                              ssems[h]).wait()

    start_gathers(0, 0)
    for g in range(NGROUP):
        h = g % 2
        if g + 1 < NGROUP:
            if g >= 1:
                wait_scatter(1 - h)
            start_gathers(g + 1, 1 - h)
        wait_gathers(h)
        start_scatter(g, h)
    wait_scatter(0)
    wait_scatter(1)


def kernel(indices, table):
    idx_flat = indices.astype(jnp.int32).reshape(TOTAL)
    out = _gather_sc(idx_flat, table)
    return out.reshape(BATCH, N_FIELDS, DIM)


# speedup vs baseline: 1.2216x; 1.0036x over previous
import functools

import jax
import jax.numpy as jnp
from jax import lax
from jax.experimental import pallas as pl
from jax.experimental.pallas import tpu as pltpu
from jax.experimental.pallas import tpu_sc as plsc

NUM_EMB = 1_000_000
DIM = 32
BATCH = 16384
N_FIELDS = 26
TOTAL = BATCH * N_FIELDS

NC = 2
NS = 16
NW = NC * NS
IDX_PER_TILE = TOTAL // NW
CHUNK = 128
G = 1664
NCHUNK = G // CHUNK
NGROUP = IDX_PER_TILE // G

_mesh = plsc.VectorSubcoreMesh(core_axis_name="c", subcore_axis_name="s")


@functools.partial(
    pl.kernel,
    mesh=_mesh,
    out_type=jax.ShapeDtypeStruct((TOTAL, DIM), jnp.float32),
    compiler_params=pltpu.CompilerParams(use_tc_tiling_on_sc=False),
    scratch_types=[
        pltpu.VMEM((IDX_PER_TILE,), jnp.int32),
        pltpu.VMEM((2, G, DIM), jnp.float32),
        pltpu.SemaphoreType.DMA,
        pltpu.SemaphoreType.DMA,
        pltpu.SemaphoreType.DMA,
        pltpu.SemaphoreType.DMA,
    ],
)
def _gather_sc(idx_hbm, table_hbm, out_hbm, idx_v, buf, gsem0, gsem1,
               ssem0, ssem1):
    wid = lax.axis_index("s") * NC + lax.axis_index("c")
    base = wid * IDX_PER_TILE
    gsems = (gsem0, gsem1)
    ssems = (ssem0, ssem1)

    pltpu.sync_copy(idx_hbm.at[pl.ds(base, IDX_PER_TILE)], idx_v)

    def start_gathers(g, h):
        def body(c, carry):
            pltpu.async_copy(
                table_hbm.at[idx_v.at[pl.ds(g * G + c * CHUNK, CHUNK)]],
                buf.at[h].at[pl.ds(c * CHUNK, CHUNK)],
                gsems[h],
            )
            return carry
        lax.fori_loop(0, NCHUNK, body, 0)

    def wait_gathers(h):
        pltpu.make_async_copy(out_hbm.at[pl.ds(0, G)], buf.at[h],
                              gsems[h]).wait()

    def start_scatter(g, h):
        pltpu.async_copy(buf.at[h], out_hbm.at[pl.ds(base + g * G, G)],
                         ssems[h])

    def wait_scatter(h):
        pltpu.make_async_copy(buf.at[h], out_hbm.at[pl.ds(0, G)],
                              ssems[h]).wait()

    start_gathers(0, 0)
    for g in range(NGROUP):
        h = g % 2
        if g + 1 < NGROUP:
            if g >= 1:
                wait_scatter(1 - h)
            start_gathers(g + 1, 1 - h)
        wait_gathers(h)
        start_scatter(g, h)
    wait_scatter(0)
    wait_scatter(1)


def kernel(indices, table):
    idx_flat = lax.optimization_barrier(
        indices.astype(jnp.int32).reshape(TOTAL))
    tab_flat = lax.optimization_barrier(table.reshape(NUM_EMB * DIM))
    out = _gather_sc(idx_flat, tab_flat.reshape(NUM_EMB, DIM))
    out_flat = lax.optimization_barrier(out.reshape(TOTAL * DIM))
    return out_flat.reshape(BATCH, N_FIELDS, DIM)
